# baseline (speedup 1.0000x reference)
import jax
import jax.numpy as jnp
from jax import lax
from jax.experimental import pallas as pl
from jax.experimental.pallas import tpu as pltpu
from jax.experimental.pallas import tpu_sc as plsc

S, HW, B, C = 16, 256, 16, 64
DEC = 16
N_TOK = S * HW
N_ROWS = N_TOK * B
OUT_ROWS = DEC * B + N_ROWS

_info = plsc.get_sparse_core_info()
NC, NS = _info.num_cores, _info.num_subcores
NW = NC * NS
ROWS_W = N_ROWS // NW
TOK_W = N_TOK // NW
CHUNK = 512
N_CHUNK = ROWS_W // CHUNK
G = 128
N_G = CHUNK // G
TOK_CHUNK = CHUNK // B


def _body(x_hbm, emb_hbm, dec_hbm, pos_hbm, out_hbm,
          idx_v, pos_v, rows_v, dec_idx_v, dec_rows_v, sem, dec_sem):
    wid = lax.axis_index("s") * NC + lax.axis_index("c")
    row0 = wid * ROWS_W
    tok0 = wid * TOK_W

    pltpu.sync_copy(x_hbm.at[pl.ds(row0, ROWS_W)], idx_v)
    pltpu.sync_copy(pos_hbm.at[pl.ds(tok0, TOK_W)], pos_v)

    for c in range(N_CHUNK):
        copies = []
        for g in range(N_G):
            src = emb_hbm.at[idx_v.at[pl.ds(c * CHUNK + g * G, G)]]
            dst = rows_v.at[pl.ds(g * G, G)]
            copies.append(pltpu.async_copy(src, dst, sem))
        for cp in copies:
            cp.wait()

        def tok_add(j, _):
            for k in range(C // 16):
                p = pos_v[c * TOK_CHUNK + j, pl.ds(k * 16, 16)]
                for b in range(B):
                    plsc.addupdate(rows_v.at[j * B + b, pl.ds(k * 16, 16)], p)
            return 0

        lax.fori_loop(0, TOK_CHUNK, tok_add, 0)

        pltpu.sync_copy(
            rows_v,
            out_hbm.at[pl.ds(DEC * B + row0 + c * CHUNK, CHUNK)])

    @pl.when(wid == 0)
    def _dec():
        for i in range(DEC):
            dec_idx_v[pl.ds(i * B, B)] = jnp.full((B,), i, jnp.int32)
        pltpu.async_copy(dec_hbm.at[dec_idx_v], dec_rows_v, dec_sem).wait()
        pltpu.sync_copy(dec_rows_v, out_hbm.at[pl.ds(0, DEC * B)])


@jax.jit
def _embed(x_flat, emb_table, dec_table, pos2d):
    mesh = plsc.VectorSubcoreMesh(core_axis_name="c", subcore_axis_name="s")
    out = pl.kernel(
        _body,
        out_type=jax.ShapeDtypeStruct((OUT_ROWS, C), jnp.float32),
        mesh=mesh,
        compiler_params=pltpu.CompilerParams(use_tc_tiling_on_sc=False),
        scratch_types=[
            pltpu.VMEM((ROWS_W,), jnp.int32),
            pltpu.VMEM((TOK_W, C), jnp.float32),
            pltpu.VMEM((CHUNK, C), jnp.float32),
            pltpu.VMEM((DEC * B,), jnp.int32),
            pltpu.VMEM((DEC * B, C), jnp.float32),
            pltpu.SemaphoreType.DMA,
            pltpu.SemaphoreType.DMA,
        ],
    )(x_flat, emb_table, dec_table, pos2d)
    return out


def kernel(x, emb_table, dec_table, pos_encoding):
    x_flat = x.reshape(-1).astype(jnp.int32)
    pos2d = pos_encoding.reshape(N_TOK, C)
    out = _embed(x_flat, emb_table, dec_table, pos2d)
    return out.reshape(DEC + N_TOK, B, C)
